# baseline (speedup 1.0000x reference)
import jax
import jax.numpy as jnp
from jax import lax
from jax.experimental import pallas as pl
from jax.experimental.pallas import tpu as pltpu
from jax.experimental.pallas import tpu_sc as plsc

_N = 5_000_000
_B = 1_000_000

_NC = 2
_NS = 16
_NW = _NC * _NS
_CHUNK = 31_360
_S = 3_920
_NSUB = _CHUNK // _S


def _body(packed_hbm, idx_hbm, out_rows,
          idx_v, rows_a, rows_b, sem_a, sem_b, sem_oa, sem_ob):
    wid = lax.axis_index("s") * _NC + lax.axis_index("c")
    base = lax.min(wid * _CHUNK, _B - _CHUNK)
    pltpu.sync_copy(idx_hbm.at[pl.ds(base, _CHUNK)], idx_v)

    rows = (rows_a, rows_b)
    sems = (sem_a, sem_b)
    osems = (sem_oa, sem_ob)

    def fire(k):
        return pltpu.async_copy(
            packed_hbm.at[idx_v.at[pl.ds(k * _S, _S)]], rows[k % 2], sems[k % 2])

    h = [fire(0)]
    out_h = [None, None]
    for k in range(_NSUB):
        if k + 1 < _NSUB:
            h.append(fire(k + 1))
        h[k].wait()
        if out_h[k % 2] is not None:
            out_h[k % 2].wait()
        out_h[k % 2] = pltpu.async_copy(
            rows[k % 2], out_rows.at[pl.ds(base + k * _S, _S), :], osems[k % 2])
    for oh in out_h:
        if oh is not None:
            oh.wait()


def kernel(src, dst, edge_idxs, timestamps, idx):
    ts_i = lax.bitcast_convert_type(timestamps, jnp.int32)
    packed = jnp.transpose(jnp.stack(
        [src, dst, edge_idxs, ts_i, src, src, src, src], axis=0))

    call = pl.kernel(
        _body,
        out_type=jax.ShapeDtypeStruct((_B, 8), jnp.int32),
        mesh=plsc.VectorSubcoreMesh(core_axis_name="c", subcore_axis_name="s"),
        compiler_params=pltpu.CompilerParams(use_tc_tiling_on_sc=False),
        scratch_types=[
            pltpu.VMEM((_CHUNK,), jnp.int32),
            pltpu.VMEM((_S, 8), jnp.int32),
            pltpu.VMEM((_S, 8), jnp.int32),
            pltpu.SemaphoreType.DMA,
            pltpu.SemaphoreType.DMA,
            pltpu.SemaphoreType.DMA,
            pltpu.SemaphoreType.DMA,
        ],
    )
    out_rows = call(packed, idx)
    return (out_rows[:, 0], out_rows[:, 1], out_rows[:, 2],
            lax.bitcast_convert_type(out_rows[:, 3], jnp.float32))


# speedup vs baseline: 25.8761x; 25.8761x over previous
import jax
import jax.numpy as jnp
from jax import lax
from jax.experimental import pallas as pl
from jax.experimental.pallas import tpu as pltpu
from jax.experimental.pallas import tpu_sc as plsc

_N = 5_000_000
_B = 1_000_000

_NC = 2
_NS = 16
_NW = _NC * _NS
_CHUNK = 31_360


def _body(src_hbm, dst_hbm, edge_hbm, ts_hbm, idx_hbm,
          out_s, out_d, out_e, out_t,
          idx_v, buf0, buf1, buft, sem0, sem1, semt):
    wid = lax.axis_index("s") * _NC + lax.axis_index("c")
    base = lax.min(wid * _CHUNK, _B - _CHUNK)
    pltpu.sync_copy(idx_hbm.at[pl.ds(base, _CHUNK)], idx_v)

    h0 = pltpu.async_copy(src_hbm.at[idx_v], buf0, sem0)
    h1 = pltpu.async_copy(dst_hbm.at[idx_v], buf1, sem1)
    ht = pltpu.async_copy(ts_hbm.at[idx_v], buft, semt)
    h0.wait()
    pltpu.sync_copy(buf0, out_s.at[pl.ds(base, _CHUNK)])
    h2 = pltpu.async_copy(edge_hbm.at[idx_v], buf0, sem0)
    h1.wait()
    pltpu.sync_copy(buf1, out_d.at[pl.ds(base, _CHUNK)])
    h2.wait()
    pltpu.sync_copy(buf0, out_e.at[pl.ds(base, _CHUNK)])
    ht.wait()
    pltpu.sync_copy(buft, out_t.at[pl.ds(base, _CHUNK)])


def kernel(src, dst, edge_idxs, timestamps, idx):
    i32_out = jax.ShapeDtypeStruct((_B,), jnp.int32)
    f32_out = jax.ShapeDtypeStruct((_B,), jnp.float32)
    call = pl.kernel(
        _body,
        out_type=(i32_out, i32_out, i32_out, f32_out),
        mesh=plsc.VectorSubcoreMesh(core_axis_name="c", subcore_axis_name="s"),
        scratch_types=[
            pltpu.VMEM((_CHUNK,), jnp.int32),
            pltpu.VMEM((_CHUNK,), jnp.int32),
            pltpu.VMEM((_CHUNK,), jnp.int32),
            pltpu.VMEM((_CHUNK,), jnp.float32),
            pltpu.SemaphoreType.DMA,
            pltpu.SemaphoreType.DMA,
            pltpu.SemaphoreType.DMA,
        ],
    )
    return call(src, dst, edge_idxs, timestamps, idx)


# speedup vs baseline: 26.0210x; 1.0056x over previous
import jax
import jax.numpy as jnp
from jax import lax
from jax.experimental import pallas as pl
from jax.experimental.pallas import tpu as pltpu
from jax.experimental.pallas import tpu_sc as plsc

_N = 5_000_000
_B = 1_000_000

_NC = 2
_NS = 16
_NW = _NC * _NS
_CHUNK = 31_360
_H = _CHUNK // 2


def _body(src_hbm, dst_hbm, edge_hbm, ts_hbm, idx_hbm,
          out_s, out_d, out_e, out_t,
          idx_v, ba, bb, bc, bd, t0, t1,
          sa, sb, sc, sd, st0, st1):
    wid = lax.axis_index("s") * _NC + lax.axis_index("c")
    base = lax.min(wid * _CHUNK, _B - _CHUNK)

    def gather(table, half, buf, sem):
        return pltpu.async_copy(
            table.at[idx_v.at[pl.ds(half * _H, _H)]], buf, sem)

    def copyout(buf, out, half):
        pltpu.sync_copy(buf, out.at[pl.ds(base + half * _H, _H)])

    pltpu.sync_copy(idx_hbm.at[pl.ds(base, _H)], idx_v.at[pl.ds(0, _H)])
    h_s0 = gather(src_hbm, 0, ba, sa)
    h_d0 = gather(dst_hbm, 0, bb, sb)
    h_e0 = gather(edge_hbm, 0, bc, sc)
    h_t0 = gather(ts_hbm, 0, t0, st0)
    pltpu.sync_copy(idx_hbm.at[pl.ds(base + _H, _H)], idx_v.at[pl.ds(_H, _H)])
    h_s1 = gather(src_hbm, 1, bd, sd)
    h_t1 = gather(ts_hbm, 1, t1, st1)

    h_s0.wait()
    copyout(ba, out_s, 0)
    h_d1 = gather(dst_hbm, 1, ba, sa)
    h_d0.wait()
    copyout(bb, out_d, 0)
    h_e1 = gather(edge_hbm, 1, bb, sb)
    h_e0.wait()
    copyout(bc, out_e, 0)
    h_t0.wait()
    copyout(t0, out_t, 0)
    h_s1.wait()
    copyout(bd, out_s, 1)
    h_d1.wait()
    copyout(ba, out_d, 1)
    h_e1.wait()
    copyout(bb, out_e, 1)
    h_t1.wait()
    copyout(t1, out_t, 1)


def kernel(src, dst, edge_idxs, timestamps, idx):
    i32_out = jax.ShapeDtypeStruct((_B,), jnp.int32)
    f32_out = jax.ShapeDtypeStruct((_B,), jnp.float32)
    call = pl.kernel(
        _body,
        out_type=(i32_out, i32_out, i32_out, f32_out),
        mesh=plsc.VectorSubcoreMesh(core_axis_name="c", subcore_axis_name="s"),
        scratch_types=[
            pltpu.VMEM((_CHUNK,), jnp.int32),
            pltpu.VMEM((_H,), jnp.int32),
            pltpu.VMEM((_H,), jnp.int32),
            pltpu.VMEM((_H,), jnp.int32),
            pltpu.VMEM((_H,), jnp.int32),
            pltpu.VMEM((_H,), jnp.float32),
            pltpu.VMEM((_H,), jnp.float32),
            pltpu.SemaphoreType.DMA,
            pltpu.SemaphoreType.DMA,
            pltpu.SemaphoreType.DMA,
            pltpu.SemaphoreType.DMA,
            pltpu.SemaphoreType.DMA,
            pltpu.SemaphoreType.DMA,
        ],
    )
    return call(src, dst, edge_idxs, timestamps, idx)
